# baseline (device time: 57778 ns/iter reference)
import jax
import jax.numpy as jnp
from jax import lax
from jax.experimental import pallas as pl
from jax.experimental.pallas import tpu as pltpu

B = 32
NB = 256
BS = 32
H = 16
HH = H // 2
D = 128
QP = 64
KK = QP * BS
SCALE = D ** -0.5
NEG = -1e30


def kernel(Q, K, V, bt, lens):
    lens2 = lens.reshape(B, 1)
    qidx = (2 * lax.axis_index("y") + lax.axis_index("z")).reshape(1)

    def body(qidx_ref, q_ref, k_ref, v_ref, bt_ref, lens_ref, out_ref,
             kbf_ref, vbf_ref, acc_ref, abf_ref, stats_ref,
             racc_ref, rstats_ref, sems):
        mx = lax.axis_index("x")
        my = lax.axis_index("y")
        mz = lax.axis_index("z")
        peers = [(mx, my, 1 - mz), (mx, 1 - my, mz), (1 - mx, my, mz)]

        bar = pltpu.get_barrier_semaphore()
        for peer in peers:
            pl.semaphore_signal(bar, inc=1, device_id=peer,
                                device_id_type=pl.DeviceIdType.MESH)

        kbf_ref[...] = k_ref[...].astype(jnp.bfloat16)
        vbf_ref[...] = v_ref[...].astype(jnp.bfloat16)

        gid0 = mx * NB + qidx_ref[0] * QP
        page_ids = gid0 + lax.broadcasted_iota(jnp.int32, (1, 1, QP), 2)
        btv = bt_ref[...]
        jidx = lax.broadcasted_iota(jnp.int32, (1, NB, 1), 1)
        valid = jidx < lens_ref[...].reshape(B, 1, 1)
        eq = (btv[:, :, None] == page_ids) & valid
        w = jnp.sum(eq.astype(jnp.float32), axis=1)
        lw = jnp.log(w)
        lw_keys = jnp.broadcast_to(lw[:, :, None], (B, QP, BS)).reshape(B, KK)

        def compute_half(half):
            for hh in range(HH):
                h = half * HH + hh
                q = q_ref[:, 0, h, :].astype(jnp.bfloat16)
                k = kbf_ref[:, :, h, :].reshape(KK, D)
                s = lax.dot_general(q, k, (((1,), (1,)), ((), ())),
                                    preferred_element_type=jnp.float32)
                s = s * SCALE + lw_keys
                m_h = jnp.maximum(jnp.max(s, axis=1, keepdims=True), NEG)
                p = jnp.exp(s - m_h)
                v = vbf_ref[:, :, h, :].reshape(KK, D)
                pv = lax.dot_general(p.astype(jnp.bfloat16), v,
                                     (((1,), (0,)), ((), ())),
                                     preferred_element_type=jnp.float32)
                acc_ref[half, hh] = pv
                stats_ref[half, 0, hh:hh + 1, :] = m_h.T
                stats_ref[half, 1, hh:hh + 1, :] = jnp.sum(
                    p, axis=1, keepdims=True).T

        def start_stage(s_i, half):
            base = 8 * s_i + 4 * half
            rd_s = pltpu.make_async_remote_copy(
                src_ref=stats_ref.at[half], dst_ref=rstats_ref.at[s_i, half],
                send_sem=sems.at[base + 2], recv_sem=sems.at[base + 3],
                device_id=peers[s_i], device_id_type=pl.DeviceIdType.MESH)
            rd_s.start()
            abf_ref[half] = acc_ref[half].astype(jnp.bfloat16)
            rd_a = pltpu.make_async_remote_copy(
                src_ref=abf_ref.at[half], dst_ref=racc_ref.at[s_i, half],
                send_sem=sems.at[base], recv_sem=sems.at[base + 1],
                device_id=peers[s_i], device_id_type=pl.DeviceIdType.MESH)
            rd_a.start()
            return rd_a, rd_s

        def finish_stage(rds, s_i, half):
            rd_a, rd_s = rds
            rd_a.wait()
            rd_s.wait()
            m_mine = stats_ref[half, 0]
            l_mine = stats_ref[half, 1]
            m_peer = rstats_ref[s_i, half, 0]
            l_peer = rstats_ref[s_i, half, 1]
            m_tot = jnp.maximum(m_mine, m_peer)
            a = jnp.exp(m_mine - m_tot)
            b = jnp.exp(m_peer - m_tot)
            stats_ref[half, 0] = m_tot
            stats_ref[half, 1] = l_mine * a + l_peer * b
            acc_ref[half] = (acc_ref[half] * a[:, :, None]
                             + racc_ref[s_i, half].astype(jnp.float32)
                             * b[:, :, None])

        compute_half(0)
        pl.semaphore_wait(bar, 3)
        rds_a = start_stage(0, 0)
        compute_half(1)
        rds_b = start_stage(0, 1)
        for s_i in range(3):
            finish_stage(rds_a, s_i, 0)
            if s_i < 2:
                rds_a = start_stage(s_i + 1, 0)
            finish_stage(rds_b, s_i, 1)
            if s_i < 2:
                rds_b = start_stage(s_i + 1, 1)

        l_all = stats_ref[...][:, 1].reshape(H, B)
        final = acc_ref[...].reshape(H, B, D) / l_all[:, :, None]
        out_ref[...] = jnp.transpose(final, (1, 0, 2)).reshape(B, 1, H, D)

    grid_spec = pltpu.PrefetchScalarGridSpec(
        num_scalar_prefetch=1,
        grid=(1,),
        in_specs=[
            pl.BlockSpec((B, 1, H, D), lambda i, s: (0, 0, 0, 0)),
            pl.BlockSpec((QP, BS, H, D), lambda i, s: (s[0], 0, 0, 0)),
            pl.BlockSpec((QP, BS, H, D), lambda i, s: (s[0], 0, 0, 0)),
            pl.BlockSpec((B, NB), lambda i, s: (0, 0)),
            pl.BlockSpec((B, 1), lambda i, s: (0, 0)),
        ],
        out_specs=pl.BlockSpec((B, 1, H, D), lambda i, s: (0, 0, 0, 0)),
        scratch_shapes=[
            pltpu.VMEM((QP, BS, H, D), jnp.bfloat16),
            pltpu.VMEM((QP, BS, H, D), jnp.bfloat16),
            pltpu.VMEM((2, HH, B, D), jnp.float32),
            pltpu.VMEM((2, HH, B, D), jnp.bfloat16),
            pltpu.VMEM((2, 2, HH, B), jnp.float32),
            pltpu.VMEM((3, 2, HH, B, D), jnp.bfloat16),
            pltpu.VMEM((3, 2, 2, HH, B), jnp.float32),
            pltpu.SemaphoreType.DMA((24,)),
        ],
    )

    return pl.pallas_call(
        body,
        grid_spec=grid_spec,
        out_shape=jax.ShapeDtypeStruct((B, 1, H, D), jnp.float32),
        compiler_params=pltpu.CompilerParams(
            collective_id=0,
            vmem_limit_bytes=100 * 1024 * 1024,
        ),
    )(qidx, Q, K, V, bt, lens2)
